# baseline (speedup 1.0000x reference)
import jax
import jax.numpy as jnp
from jax import lax
from jax.experimental import pallas as pl
from jax.experimental.pallas import tpu as pltpu
from jax.experimental.pallas import tpu_sc as plsc

V = 100000
D = 64
B = 4096
L = 50
NC, NS = 2, 16
NW = NC * NS
RPW = B // NW
CH = 2
TPC = CH * L
NCH = RPW // CH
G = 16
NG = RPW // G
CPG = G // CH
KD = D // 16
LP = 64
NBUF = CPG


def _body(table, wts, qidx, out, idx_v, w_v, coef_v, rows_v,
          out_v, sem_w, sems):
    wid = lax.axis_index("s") * NC + lax.axis_index("c")

    pltpu.sync_copy(qidx.at[pl.ds(wid * NCH, NCH)], idx_v)

    for j in range(2 * CPG):
        pltpu.async_copy(wts.at[idx_v.at[j]], w_v.at[j], sem_w)
    for b in range(NBUF):
        pltpu.async_copy(table.at[idx_v.at[b]], rows_v.at[b], sems.at[b])

    lane = lax.iota(jnp.int32, 16)
    half = lane // CH
    colbase = (lane % CH) * L

    def group(g, carry):
        @pl.when(g < NG - 2)
        def _():
            for j in range(CPG):
                c = (g + 2) * CPG + j
                pltpu.async_copy(wts.at[idx_v.at[c]], w_v.at[c], sem_w)

        for j in range(CPG):
            c = g * CPG + j
            pltpu.make_async_copy(wts.at[idx_v.at[c]], w_v.at[c],
                                  sem_w).wait()

        row2 = g * (G // CH) + half
        rvec = g * G + lane
        m = jnp.full((16,), -jnp.inf, jnp.float32)
        for l in range(L):
            wv = plsc.load_gather(w_v, [row2, colbase + l])
            m = jnp.maximum(m, wv)
        s = jnp.zeros((16,), jnp.float32)
        for l in range(L):
            wv = plsc.load_gather(w_v, [row2, colbase + l])
            e = jnp.exp(wv - m)
            s = s + e
            plsc.store_scatter(coef_v, [rvec, jnp.full((16,), l, jnp.int32)], e)
        inv = 1.0 / s
        for l in range(L):
            lv = jnp.full((16,), l, jnp.int32)
            cv = plsc.load_gather(coef_v, [rvec, lv])
            plsc.store_scatter(coef_v, [rvec, lv], cv * inv)

        for j in range(CPG):
            c = g * CPG + j
            pltpu.make_async_copy(
                table.at[idx_v.at[c]], rows_v.at[j], sems.at[j]).wait()

            def chunk_row(r2, inner):
                rg = c * CH + r2
                tok0 = r2 * L
                acc = [jnp.zeros((16,), jnp.float32) for _ in range(KD)]
                for base in range(0, L, 16):
                    cv = coef_v[rg, pl.ds(base, 16)]
                    for jj in range(min(16, L - base)):
                        cs = cv[jj]
                        for k in range(KD):
                            acc[k] = acc[k] + cs * rows_v[j, tok0 + base + jj,
                                                          pl.ds(k * 16, 16)]
                for k in range(KD):
                    out_v[rg, pl.ds(k * 16, 16)] = acc[k]
                return inner
            lax.fori_loop(0, CH, chunk_row, 0)

            @pl.when(g < NG - 1)
            def _():
                pltpu.async_copy(
                    table.at[idx_v.at[c + NBUF]], rows_v.at[j], sems.at[j])
        return carry
    lax.fori_loop(0, NG, group, 0)

    pltpu.sync_copy(out_v, out.at[pl.ds(wid * RPW, RPW)])


@jax.jit
def kernel(table, weights, query):
    qidx = query.astype(jnp.int32).reshape(NW * NCH, TPC)
    w1 = weights.reshape(V)
    mesh = plsc.VectorSubcoreMesh(core_axis_name="c", subcore_axis_name="s")
    k = pl.kernel(
        _body,
        out_type=jax.ShapeDtypeStruct((B, D), jnp.float32),
        mesh=mesh,
        scratch_types=[
            pltpu.VMEM((NCH, TPC), jnp.int32),
            pltpu.VMEM((NCH, TPC), jnp.float32),
            pltpu.VMEM((RPW, LP), jnp.float32),
            pltpu.VMEM((NBUF, TPC, D), jnp.float32),
            pltpu.VMEM((RPW, D), jnp.float32),
            pltpu.SemaphoreType.DMA,
            pltpu.SemaphoreType.DMA((NBUF,)),
        ],
        compiler_params=pltpu.CompilerParams(
            use_tc_tiling_on_sc=False, needs_layout_passes=False),
    )
    return k(table, w1, qidx)


# speedup vs baseline: 1.1229x; 1.1229x over previous
import jax
import jax.numpy as jnp
from jax import lax
from jax.experimental import pallas as pl
from jax.experimental.pallas import tpu as pltpu
from jax.experimental.pallas import tpu_sc as plsc

V = 100000
D = 64
B = 4096
L = 50
NC, NS = 2, 16
NW = NC * NS
RPW = B // NW
CH = 8
TPC = CH * L
NCH = RPW // CH
G = 16
NG = RPW // G
CPG = G // CH
KD = D // 16
LP = 64
NBUF = CPG


def _body(table, wts, qidx, out, idx_v, w_v, coef_v, rows_v,
          out_v, sem_w, sems):
    wid = lax.axis_index("s") * NC + lax.axis_index("c")

    pltpu.sync_copy(qidx.at[pl.ds(wid * NCH, NCH)], idx_v)

    for j in range(2 * CPG):
        pltpu.async_copy(wts.at[idx_v.at[j]], w_v.at[j], sem_w)
    for b in range(NBUF):
        pltpu.async_copy(table.at[idx_v.at[b]], rows_v.at[b], sems.at[b])

    lane = lax.iota(jnp.int32, 16)
    half = lane // CH
    colbase = (lane % CH) * L

    def group(g, carry):
        @pl.when(g < NG - 2)
        def _():
            for j in range(CPG):
                c = (g + 2) * CPG + j
                pltpu.async_copy(wts.at[idx_v.at[c]], w_v.at[c], sem_w)

        for j in range(CPG):
            c = g * CPG + j
            pltpu.make_async_copy(wts.at[idx_v.at[c]], w_v.at[c],
                                  sem_w).wait()

        row2 = g * (G // CH) + half
        rvec = g * G + lane
        m = jnp.full((16,), -jnp.inf, jnp.float32)
        for l in range(L):
            wv = plsc.load_gather(w_v, [row2, colbase + l])
            m = jnp.maximum(m, wv)
        s = jnp.zeros((16,), jnp.float32)
        for l in range(L):
            wv = plsc.load_gather(w_v, [row2, colbase + l])
            e = jnp.exp(wv - m)
            s = s + e
            plsc.store_scatter(coef_v, [rvec, jnp.full((16,), l, jnp.int32)], e)
        inv = 1.0 / s
        for l in range(L):
            lv = jnp.full((16,), l, jnp.int32)
            cv = plsc.load_gather(coef_v, [rvec, lv])
            plsc.store_scatter(coef_v, [rvec, lv], cv * inv)

        for j in range(CPG):
            c = g * CPG + j
            pltpu.make_async_copy(
                table.at[idx_v.at[c]], rows_v.at[j], sems.at[j]).wait()

            def chunk_row(r2, inner):
                rg = c * CH + r2
                tok0 = r2 * L
                acc = [jnp.zeros((16,), jnp.float32) for _ in range(KD)]
                for base in range(0, L, 16):
                    cv = coef_v[rg, pl.ds(base, 16)]
                    for jj in range(min(16, L - base)):
                        cs = cv[jj]
                        for k in range(KD):
                            acc[k] = acc[k] + cs * rows_v[j, tok0 + base + jj,
                                                          pl.ds(k * 16, 16)]
                for k in range(KD):
                    out_v[rg, pl.ds(k * 16, 16)] = acc[k]
                return inner
            lax.fori_loop(0, CH, chunk_row, 0)

            @pl.when(g < NG - 1)
            def _():
                pltpu.async_copy(
                    table.at[idx_v.at[c + NBUF]], rows_v.at[j], sems.at[j])
        return carry
    lax.fori_loop(0, NG, group, 0)

    pltpu.sync_copy(out_v, out.at[pl.ds(wid * RPW, RPW)])


@jax.jit
def kernel(table, weights, query):
    qidx = query.astype(jnp.int32).reshape(NW * NCH, TPC)
    w1 = weights.reshape(V)
    mesh = plsc.VectorSubcoreMesh(core_axis_name="c", subcore_axis_name="s")
    k = pl.kernel(
        _body,
        out_type=jax.ShapeDtypeStruct((B, D), jnp.float32),
        mesh=mesh,
        scratch_types=[
            pltpu.VMEM((NCH, TPC), jnp.int32),
            pltpu.VMEM((NCH, TPC), jnp.float32),
            pltpu.VMEM((RPW, LP), jnp.float32),
            pltpu.VMEM((NBUF, TPC, D), jnp.float32),
            pltpu.VMEM((RPW, D), jnp.float32),
            pltpu.SemaphoreType.DMA,
            pltpu.SemaphoreType.DMA((NBUF,)),
        ],
        compiler_params=pltpu.CompilerParams(
            use_tc_tiling_on_sc=False, needs_layout_passes=False),
    )
    return k(table, w1, qidx)


# speedup vs baseline: 1.2195x; 1.0860x over previous
import jax
import jax.numpy as jnp
from jax import lax
from jax.experimental import pallas as pl
from jax.experimental.pallas import tpu as pltpu
from jax.experimental.pallas import tpu_sc as plsc

V = 100000
D = 64
B = 4096
L = 50
NC, NS = 2, 16
NW = NC * NS
RPW = B // NW
CH = 8
TPC = CH * L
NCH = RPW // CH
G = 16
NG = RPW // G
CPG = G // CH
KD = D // 16
LP = 64
NBUF = 2


def _coef_body(wts, qidx, coef_out, idx_v, w_v, coef_v, sem_w):
    wid = lax.axis_index("s") * NC + lax.axis_index("c")

    pltpu.sync_copy(qidx.at[pl.ds(wid * NCH, NCH)], idx_v)

    for c in range(NCH):
        pltpu.async_copy(wts.at[idx_v.at[c]], w_v.at[c], sem_w)

    lane = lax.iota(jnp.int32, 16)
    half = lane // CH
    colbase = (lane % CH) * L

    def group(g, carry):
        for j in range(CPG):
            c = g * CPG + j
            pltpu.make_async_copy(wts.at[idx_v.at[c]], w_v.at[c],
                                  sem_w).wait()
        row2 = g * (G // CH) + half
        rvec = g * G + lane
        m = jnp.full((16,), -jnp.inf, jnp.float32)
        for l in range(L):
            wv = plsc.load_gather(w_v, [row2, colbase + l])
            m = jnp.maximum(m, wv)
        s = jnp.zeros((16,), jnp.float32)
        for l in range(L):
            wv = plsc.load_gather(w_v, [row2, colbase + l])
            e = jnp.exp(wv - m)
            s = s + e
            plsc.store_scatter(coef_v, [rvec, jnp.full((16,), l, jnp.int32)], e)
        inv = 1.0 / s
        for l in range(L):
            lv = jnp.full((16,), l, jnp.int32)
            cv = plsc.load_gather(coef_v, [rvec, lv])
            plsc.store_scatter(coef_v, [rvec, lv], cv * inv)
        return carry
    lax.fori_loop(0, NG, group, 0)

    pltpu.sync_copy(coef_v, coef_out.at[pl.ds(wid * RPW, RPW)])


def _pool_body(table, qidx, coef, out, idx_v, coef_v, rows_v, out_v, sems):
    wid = lax.axis_index("s") * NC + lax.axis_index("c")

    pltpu.sync_copy(qidx.at[pl.ds(wid * NCH, NCH)], idx_v)

    for b in range(NBUF):
        pltpu.async_copy(table.at[idx_v.at[b]], rows_v.at[b], sems.at[b])

    pltpu.sync_copy(coef.at[pl.ds(wid * RPW, RPW)], coef_v)

    def outer(o, carry):
        for b in range(NBUF):
            c = o * NBUF + b
            pltpu.make_async_copy(
                table.at[idx_v.at[c]], rows_v.at[b], sems.at[b]).wait()

            def chunk_row(r2, inner):
                rg = c * CH + r2
                tok0 = r2 * L
                acc = [jnp.zeros((16,), jnp.float32) for _ in range(KD)]
                for base in range(0, L, 16):
                    cv = coef_v[rg, pl.ds(base, 16)]
                    for jj in range(min(16, L - base)):
                        cs = cv[jj]
                        for k in range(KD):
                            acc[k] = acc[k] + cs * rows_v[b, tok0 + base + jj,
                                                          pl.ds(k * 16, 16)]
                for k in range(KD):
                    out_v[rg, pl.ds(k * 16, 16)] = acc[k]
                return inner
            lax.fori_loop(0, CH, chunk_row, 0)

            @pl.when(c + NBUF < NCH)
            def _():
                pltpu.async_copy(
                    table.at[idx_v.at[c + NBUF]], rows_v.at[b], sems.at[b])
        return carry
    lax.fori_loop(0, NCH // NBUF, outer, 0)

    pltpu.sync_copy(out_v, out.at[pl.ds(wid * RPW, RPW)])


@jax.jit
def kernel(table, weights, query):
    qidx = query.astype(jnp.int32).reshape(NW * NCH, TPC)
    w1 = weights.reshape(V)
    mesh = plsc.VectorSubcoreMesh(core_axis_name="c", subcore_axis_name="s")
    params = pltpu.CompilerParams(
        use_tc_tiling_on_sc=False, needs_layout_passes=False)

    coef_call = pl.kernel(
        _coef_body,
        out_type=jax.ShapeDtypeStruct((B, LP), jnp.float32),
        mesh=mesh,
        scratch_types=[
            pltpu.VMEM((NCH, TPC), jnp.int32),
            pltpu.VMEM((NCH, TPC), jnp.float32),
            pltpu.VMEM((RPW, LP), jnp.float32),
            pltpu.SemaphoreType.DMA,
        ],
        compiler_params=params,
    )
    coef = coef_call(w1, qidx)

    pool_call = pl.kernel(
        _pool_body,
        out_type=jax.ShapeDtypeStruct((B, D), jnp.float32),
        mesh=mesh,
        scratch_types=[
            pltpu.VMEM((NCH, TPC), jnp.int32),
            pltpu.VMEM((RPW, LP), jnp.float32),
            pltpu.VMEM((NBUF, TPC, D), jnp.float32),
            pltpu.VMEM((RPW, D), jnp.float32),
            pltpu.SemaphoreType.DMA((NBUF,)),
        ],
        compiler_params=params,
    )
    return pool_call(table, qidx, coef)
